# baseline (speedup 1.0000x reference)
import numpy as np
import jax
import jax.numpy as jnp
from jax import lax
from jax.experimental import pallas as pl
from jax.experimental.pallas import tpu as pltpu

B, L, K = 8, 512, 16
NS, NV = 100, 16
ES, EV = 32, 1
N_LAYERS = 3
E = L * K


def _normalize(x, axis=-1):
    return x / jnp.sqrt(jnp.sum(x * x, axis=axis, keepdims=True) + 1e-8)


def _rbf(d, dmin, dmax, n):
    mu = jnp.linspace(dmin, dmax, n)
    sigma = (dmax - dmin) / n
    return jnp.exp(-(((d[..., None] - mu) / sigma) ** 2))


def _dihedrals(coords):
    b, l = coords.shape[0], coords.shape[1]
    X = coords.reshape(b, 3 * l, 3)
    dX = X[:, 1:] - X[:, :-1]
    U = _normalize(dX)
    u2, u1, u0 = U[:, :-2], U[:, 1:-1], U[:, 2:]
    n2 = _normalize(jnp.cross(u2, u1))
    n1 = _normalize(jnp.cross(u1, u0))
    cosD = jnp.clip(jnp.sum(n2 * n1, -1), -0.9999, 0.9999)
    D = jnp.sign(jnp.sum(u2 * n1, -1)) * jnp.arccos(cosD)
    D = jnp.pad(D, ((0, 0), (1, 2)))
    D = D.reshape(b, l, 3)
    return jnp.concatenate([jnp.cos(D), jnp.sin(D)], -1)


def _orientations(X_ca):
    fwd = _normalize(X_ca[:, 1:] - X_ca[:, :-1])
    bwd = -fwd
    fwd = jnp.pad(fwd, ((0, 0), (0, 1), (0, 0)))
    bwd = jnp.pad(bwd, ((0, 0), (1, 0), (0, 0)))
    return jnp.stack([fwd, bwd], axis=-2)


def _bd3(w):
    z = jnp.zeros_like(w)
    return jnp.concatenate([
        jnp.concatenate([w, z, z], 1),
        jnp.concatenate([z, w, z], 1),
        jnp.concatenate([z, z, w], 1)], 0)


def _flat_gvp(p):
    return [_bd3(p['Wh']), p['Ws'], p['bs'].reshape(1, -1),
            _bd3(p['Wv']), p['Wg'], p['bg'].reshape(1, -1)]


def _flatten_params(params):
    flat = []
    flat += _flat_gvp(params['embed_node'])
    flat += _flat_gvp(params['embed_edge'])
    for lp in params['layers']:
        m0 = lp['msg0']
        Wh = m0['Wh']
        flat += [_bd3(Wh[0:NV]), _bd3(Wh[NV:2 * NV]), _bd3(Wh[2 * NV:]),
                 m0['Ws'], m0['bs'].reshape(1, -1),
                 _bd3(m0['Wv']), m0['Wg'], m0['bg'].reshape(1, -1)]
        for name in ('msg1', 'msg2', 'ff0', 'ff1'):
            flat += _flat_gvp(lp[name])
        flat += [lp['ln1_g'].reshape(1, -1), lp['ln1_b'].reshape(1, -1),
                 lp['ln2_g'].reshape(1, -1), lp['ln2_b'].reshape(1, -1)]
    return flat


_NW = 12 + N_LAYERS * (8 + 4 * 6 + 4)


def _mm(a, b):
    return jnp.dot(a, b, preferred_element_type=jnp.float32)


def _mmb(a, b):
    return jnp.dot(a.astype(jnp.bfloat16), b.astype(jnp.bfloat16),
                   preferred_element_type=jnp.float32)


def _vnorm(Vh, eps):
    h = Vh.shape[-1] // 3
    v2 = Vh * Vh
    return jnp.sqrt(v2[:, :h] + v2[:, h:2 * h] + v2[:, 2 * h:] + eps)


def _gvp_k(p, s, Vt):
    Whb, Ws, bs, Wvb, Wg, bg = p
    mm = _mmb if s.shape[0] == E else _mm
    si = s.shape[-1]
    Vh = mm(Vt, Whb)
    vn = _vnorm(Vh, 1e-8)
    if vn.shape[-1] == 1:
        vterm = vn * Ws[si:si + 1]
    else:
        vterm = mm(vn, Ws[si:])
    so = mm(s, Ws[:si]) + vterm + bs
    sa = jnp.maximum(so, 0.0)
    Vv = mm(Vh, Wvb)
    gate = jax.nn.sigmoid(mm(sa, Wg) + bg)
    vo = gate.shape[-1]
    if vo > 1:
        gate = jnp.concatenate([gate, gate, gate], -1)
    return sa, Vv * gate


def _gvp_msg0(p, s, V, sgt, Vg, es, eV):
    Whd, Whs, Whe, Ws, bs, Wv, Wg, bg = p
    Wsd = Ws[0:NS]
    Wse = Ws[2 * NS:2 * NS + ES]
    Wsv = Ws[2 * NS + ES:]
    H3 = Whd.shape[1]
    u = _mm(s, Wsd) + bs
    A = _mm(V, Whd)
    Vh = ((_mmb(Vg, Whs) + _mmb(eV, Whe)).reshape(K, L, H3)
          + A[None, :, :]).reshape(E, H3)
    vn = _vnorm(Vh, 1e-8)
    so = ((sgt + _mmb(es, Wse) + _mmb(vn, Wsv)).reshape(K, L, NS)
          + u[None, :, :]).reshape(E, NS)
    sa = jnp.maximum(so, 0.0)
    Vv = _mmb(Vh, Wv)
    gate = jax.nn.sigmoid(_mmb(sa, Wg) + bg)
    gate = jnp.concatenate([gate, gate, gate], -1)
    return sa, Vv * gate


def _ln_k(s, Vt, g, b):
    mu = jnp.mean(s, -1, keepdims=True)
    var = jnp.mean((s - mu) ** 2, -1, keepdims=True)
    s = (s - mu) / jnp.sqrt(var + 1e-4) * g + b
    v2 = Vt * Vt
    vn2 = v2[:, :NV] + v2[:, NV:2 * NV] + v2[:, 2 * NV:]
    vn = jnp.sqrt(jnp.mean(vn2, -1, keepdims=True) + 1e-4)
    return s, Vt / vn


_RBF_SIG = 20.0 / 16


def _pos_freq():
    i8 = lax.broadcasted_iota(jnp.int32, (1, 8), 1).astype(jnp.float32)
    return jnp.exp(i8 * (2.0 * (-np.log(10000.0) / 16)))


def _rbf_mu():
    i16 = lax.broadcasted_iota(jnp.int32, (1, 16), 1).astype(jnp.float32)
    return i16 * (20.0 / 15.0)


def _fwd_kernel(*refs):
    (xca_ref, xcat_ref, vcol_ref, vrow_ref, sin_ref, vpk_ref) = refs[:6]
    w = [r[...] for r in refs[6:6 + _NW]]
    out_s_ref, out_v_ref = refs[6 + _NW:]

    idx = 0

    def take(n):
        nonlocal idx
        out = w[idx:idx + n]
        idx += n
        return out

    embed_node = take(6)
    embed_edge = take(6)
    layers = []
    for _ in range(N_LAYERS):
        lyr = {'msg0': take(8)}
        for n in ('msg1', 'msg2', 'ff0', 'ff1'):
            lyr[n] = take(6)
        lyr['ln1_g'], lyr['ln1_b'], lyr['ln2_g'], lyr['ln2_b'] = take(4)
        layers.append(lyr)

    X = xca_ref[...]
    Xt = xcat_ref[...]
    vcol = vcol_ref[...]
    vrow = vrow_ref[...]

    d2 = jnp.zeros((L, L), jnp.float32)
    for c in range(3):
        diff = X[:, c:c + 1] - Xt[c:c + 1, :]
        d2 = d2 + diff * diff
    ii = lax.broadcasted_iota(jnp.int32, (L, L), 0)
    jj = lax.broadcasted_iota(jnp.int32, (L, L), 1)
    d2 = jnp.where(vcol * vrow > 0, d2, 1e10) \
        + jnp.where(ii == jj, 1e12, 0.0)

    d2m = d2
    slots = []
    for _ in range(K):
        t = jnp.min(d2m, axis=1, keepdims=True)
        m = d2m == t
        slots.append(m.astype(jnp.bfloat16)[None])
        d2m = jnp.where(m, 3e38, d2m)
    Pf = jnp.concatenate(slots, axis=0).reshape(E, L)

    jall = lax.broadcasted_iota(jnp.int32, (L, 1), 0)
    jhi = (jall // 16).astype(jnp.bfloat16)
    jlo = (jall % 16).astype(jnp.bfloat16)
    Xhi = X.astype(jnp.bfloat16)
    Xlo = (X - Xhi.astype(jnp.float32)).astype(jnp.bfloat16)
    geo = jnp.concatenate([jhi, jlo, Xhi, Xlo], -1)
    gg = _mm(Pf, geo)
    src_e = gg[:, 0:1] * 16.0 + gg[:, 1:2]
    dst_e = lax.broadcasted_iota(jnp.int32, (K, L, 1), 1).astype(
        jnp.float32).reshape(E, 1)
    off = src_e - dst_e
    ang = off * _pos_freq()
    pemb = jnp.concatenate([jnp.cos(ang), jnp.sin(ang)], -1)
    Xg = gg[:, 2:5] + gg[:, 5:8]
    Xd = jnp.broadcast_to(X[None, :, :], (K, L, 3)).reshape(E, 3)
    ev = Xg - Xd
    d = jnp.sqrt(jnp.sum(ev * ev, -1, keepdims=True) + 1e-8)
    erbf = jnp.exp(-(((d - _rbf_mu()) / _RBF_SIG) ** 2))
    es = jnp.concatenate([erbf, pemb], -1)
    eV = ev / d

    s, V = _gvp_k(embed_node, sin_ref[...], vpk_ref[...])
    es, eV = _gvp_k(embed_edge, es, eV)

    for lyr in layers:
        Wss = lyr['msg0'][3][NS:2 * NS]
        sgt = jnp.dot(Pf, _mm(s, Wss).astype(jnp.bfloat16),
                      preferred_element_type=jnp.float32)
        Vg = jnp.dot(Pf, V.astype(jnp.bfloat16),
                     preferred_element_type=jnp.float32)
        ms, mV = _gvp_msg0(lyr['msg0'], s, V, sgt, Vg, es, eV)
        ms, mV = _gvp_k(lyr['msg1'], ms, mV)
        ms, mV = _gvp_k(lyr['msg2'], ms, mV)
        agg_s = jnp.sum(ms.reshape(K, L, NS), axis=0) * (1.0 / K)
        agg_V = jnp.sum(mV.reshape(K, L, 3 * NV), axis=0) * (1.0 / K)
        s, V = _ln_k(s + agg_s, V + agg_V, lyr['ln1_g'], lyr['ln1_b'])
        fs, fV = _gvp_k(lyr['ff0'], s, V)
        fs, fV = _gvp_k(lyr['ff1'], fs, fV)
        s, V = _ln_k(s + fs, V + fV, lyr['ln2_g'], lyr['ln2_b'])

    out_s_ref[...] = s
    out_v_ref[...] = V


def kernel(coords, coord_mask, padding_mask, confidence, params):
    b, l = coords.shape[0], coords.shape[1]
    X_ca = coords[:, :, 1, :]
    node_s = jnp.concatenate(
        [_dihedrals(coords), _rbf(confidence, 0.0, 1.0, 16)], -1)
    nV = _orientations(X_ca)
    vpk = jnp.concatenate([nV[..., 0], nV[..., 1], nV[..., 2]], -1)
    valid = jnp.logical_and(jnp.logical_not(padding_mask), coord_mask)
    vf = valid.astype(jnp.float32)
    vcol = vf[:, :, None]
    vrow = vf[:, None, :]
    xcat = jnp.swapaxes(X_ca, 1, 2)

    wflat = _flatten_params(params)

    def bspec(width):
        return pl.BlockSpec((None, L, width), lambda i: (i, 0, 0))

    in_specs = [
        bspec(3),
        pl.BlockSpec((None, 3, L), lambda i: (i, 0, 0)),
        bspec(1),
        pl.BlockSpec((None, 1, L), lambda i: (i, 0, 0)),
        bspec(22), bspec(6),
    ]
    for wa in wflat:
        in_specs.append(pl.BlockSpec(wa.shape, lambda i: (0, 0)))

    out_specs = [bspec(NS), bspec(3 * NV)]
    out_shape = [jax.ShapeDtypeStruct((b, l, NS), jnp.float32),
                 jax.ShapeDtypeStruct((b, l, 3 * NV), jnp.float32)]

    s, Vp = pl.pallas_call(
        _fwd_kernel,
        grid=(b,),
        in_specs=in_specs,
        out_specs=out_specs,
        out_shape=out_shape,
        compiler_params=pltpu.CompilerParams(
            vmem_limit_bytes=100 * 1024 * 1024),
    )(X_ca, xcat, vcol, vrow, node_s, vpk, *wflat)

    V = Vp.reshape(b, l, 3, NV).transpose(0, 1, 3, 2).reshape(b, l, NV * 3)
    return jnp.concatenate([s, V], -1)


# speedup vs baseline: 1.0428x; 1.0428x over previous
import numpy as np
import jax
import jax.numpy as jnp
from jax import lax
from jax.experimental import pallas as pl
from jax.experimental.pallas import tpu as pltpu

B, L, K = 8, 512, 16
NS, NV = 100, 16
ES, EV = 32, 1
N_LAYERS = 3
E = L * K


def _normalize(x, axis=-1):
    return x / jnp.sqrt(jnp.sum(x * x, axis=axis, keepdims=True) + 1e-8)


def _rbf(d, dmin, dmax, n):
    mu = jnp.linspace(dmin, dmax, n)
    sigma = (dmax - dmin) / n
    return jnp.exp(-(((d[..., None] - mu) / sigma) ** 2))


def _dihedrals(coords):
    b, l = coords.shape[0], coords.shape[1]
    X = coords.reshape(b, 3 * l, 3)
    dX = X[:, 1:] - X[:, :-1]
    U = _normalize(dX)
    u2, u1, u0 = U[:, :-2], U[:, 1:-1], U[:, 2:]
    n2 = _normalize(jnp.cross(u2, u1))
    n1 = _normalize(jnp.cross(u1, u0))
    cosD = jnp.clip(jnp.sum(n2 * n1, -1), -0.9999, 0.9999)
    D = jnp.sign(jnp.sum(u2 * n1, -1)) * jnp.arccos(cosD)
    D = jnp.pad(D, ((0, 0), (1, 2)))
    D = D.reshape(b, l, 3)
    return jnp.concatenate([jnp.cos(D), jnp.sin(D)], -1)


def _orientations(X_ca):
    fwd = _normalize(X_ca[:, 1:] - X_ca[:, :-1])
    bwd = -fwd
    fwd = jnp.pad(fwd, ((0, 0), (0, 1), (0, 0)))
    bwd = jnp.pad(bwd, ((0, 0), (1, 0), (0, 0)))
    return jnp.stack([fwd, bwd], axis=-2)


def _bd3(w):
    z = jnp.zeros_like(w)
    return jnp.concatenate([
        jnp.concatenate([w, z, z], 1),
        jnp.concatenate([z, w, z], 1),
        jnp.concatenate([z, z, w], 1)], 0)


def _flat_gvp(p):
    return [_bd3(p['Wh']), p['Ws'], p['bs'].reshape(1, -1),
            _bd3(p['Wv']), p['Wg'], p['bg'].reshape(1, -1)]


def _flatten_params(params):
    flat = []
    flat += _flat_gvp(params['embed_node'])
    flat += _flat_gvp(params['embed_edge'])
    for lp in params['layers']:
        m0 = lp['msg0']
        Wh = m0['Wh']
        flat += [_bd3(Wh[0:NV]), _bd3(Wh[NV:2 * NV]), _bd3(Wh[2 * NV:]),
                 m0['Ws'], m0['bs'].reshape(1, -1),
                 _bd3(m0['Wv']), m0['Wg'], m0['bg'].reshape(1, -1)]
        for name in ('msg1', 'msg2', 'ff0', 'ff1'):
            flat += _flat_gvp(lp[name])
        flat += [lp['ln1_g'].reshape(1, -1), lp['ln1_b'].reshape(1, -1),
                 lp['ln2_g'].reshape(1, -1), lp['ln2_b'].reshape(1, -1)]
    return flat


_NW = 12 + N_LAYERS * (8 + 4 * 6 + 4)


def _mm(a, b):
    return jnp.dot(a, b, preferred_element_type=jnp.float32)


def _mmb(a, b):
    return jnp.dot(a.astype(jnp.bfloat16), b.astype(jnp.bfloat16),
                   preferred_element_type=jnp.float32)


def _vnorm(Vh, eps):
    h = Vh.shape[-1] // 3
    v2 = Vh * Vh
    return jnp.sqrt(v2[:, :h] + v2[:, h:2 * h] + v2[:, 2 * h:] + eps)


def _gvp_k(p, s, Vt):
    Whb, Ws, bs, Wvb, Wg, bg = p
    mm = _mmb if s.shape[0] == E else _mm
    si = s.shape[-1]
    Vh = mm(Vt, Whb)
    vn = _vnorm(Vh, 1e-8)
    if vn.shape[-1] == 1:
        vterm = vn * Ws[si:si + 1]
    else:
        vterm = mm(vn, Ws[si:])
    so = mm(s, Ws[:si]) + vterm + bs
    sa = jnp.maximum(so, 0.0)
    Vv = mm(Vh, Wvb)
    gate = jax.nn.sigmoid(mm(sa, Wg) + bg)
    vo = gate.shape[-1]
    if vo > 1:
        gate = jnp.concatenate([gate, gate, gate], -1)
    return sa, Vv * gate


def _gvp_msg0(p, s, V, sgt, Vg, es, eV):
    Whd, Whs, Whe, Ws, bs, Wv, Wg, bg = p
    Wsd = Ws[0:NS]
    Wse = Ws[2 * NS:2 * NS + ES]
    Wsv = Ws[2 * NS + ES:]
    H3 = Whd.shape[1]
    u = _mm(s, Wsd) + bs
    A = _mm(V, Whd)
    Vh = ((_mmb(Vg, Whs) + _mmb(eV, Whe)).reshape(K, L, H3)
          + A[None, :, :]).reshape(E, H3)
    vn = _vnorm(Vh, 1e-8)
    so = ((sgt + _mmb(es, Wse) + _mmb(vn, Wsv)).reshape(K, L, NS)
          + u[None, :, :]).reshape(E, NS)
    sa = jnp.maximum(so, 0.0)
    Vv = _mmb(Vh, Wv)
    gate = jax.nn.sigmoid(_mmb(sa, Wg) + bg)
    gate = jnp.concatenate([gate, gate, gate], -1)
    return sa, Vv * gate


def _ln_k(s, Vt, g, b):
    mu = jnp.mean(s, -1, keepdims=True)
    var = jnp.mean((s - mu) ** 2, -1, keepdims=True)
    s = (s - mu) / jnp.sqrt(var + 1e-4) * g + b
    v2 = Vt * Vt
    vn2 = v2[:, :NV] + v2[:, NV:2 * NV] + v2[:, 2 * NV:]
    vn = jnp.sqrt(jnp.mean(vn2, -1, keepdims=True) + 1e-4)
    return s, Vt / vn


_RBF_SIG = 20.0 / 16


def _pos_freq():
    i8 = lax.broadcasted_iota(jnp.int32, (1, 8), 1).astype(jnp.float32)
    return jnp.exp(i8 * (2.0 * (-np.log(10000.0) / 16)))


def _rbf_mu():
    i16 = lax.broadcasted_iota(jnp.int32, (1, 16), 1).astype(jnp.float32)
    return i16 * (20.0 / 15.0)


def _fwd_kernel(*refs):
    (xca_ref, xcat_ref, vcol_ref, vrow_ref, sin_ref, vpk_ref) = refs[:6]
    w = [r[...] for r in refs[6:6 + _NW]]
    out_s_ref, out_v_ref = refs[6 + _NW:]

    idx = 0

    def take(n):
        nonlocal idx
        out = w[idx:idx + n]
        idx += n
        return out

    embed_node = take(6)
    embed_edge = take(6)
    layers = []
    for _ in range(N_LAYERS):
        lyr = {'msg0': take(8)}
        for n in ('msg1', 'msg2', 'ff0', 'ff1'):
            lyr[n] = take(6)
        lyr['ln1_g'], lyr['ln1_b'], lyr['ln2_g'], lyr['ln2_b'] = take(4)
        layers.append(lyr)

    X = xca_ref[...]
    Xt = xcat_ref[...]
    vcol = vcol_ref[...]
    vrow = vrow_ref[...]

    d2 = jnp.zeros((L, L), jnp.float32)
    for c in range(3):
        diff = X[:, c:c + 1] - Xt[c:c + 1, :]
        d2 = d2 + diff * diff
    ii = lax.broadcasted_iota(jnp.int32, (L, L), 0)
    jj = lax.broadcasted_iota(jnp.int32, (L, L), 1)
    d2 = jnp.where(vcol * vrow > 0, d2, 1e10) \
        + jnp.where(ii == jj, 1e12, 0.0)

    d2m = d2
    slots = []
    for _ in range(K):
        t = jnp.min(d2m, axis=1, keepdims=True)
        m = d2m == t
        slots.append(m.astype(jnp.bfloat16)[None])
        d2m = jnp.where(m, 3e38, d2m)
    Pf = jnp.concatenate(slots, axis=0).reshape(E, L)

    jall = lax.broadcasted_iota(jnp.int32, (L, 1), 0)
    jhi = (jall // 16).astype(jnp.bfloat16)
    jlo = (jall % 16).astype(jnp.bfloat16)
    src_e = (_mm(Pf, jhi) * 16.0 + _mm(Pf, jlo))
    dst_e = lax.broadcasted_iota(jnp.int32, (K, L, 1), 1).astype(
        jnp.float32).reshape(E, 1)
    off = src_e - dst_e
    ang = off * _pos_freq()
    pemb = jnp.concatenate([jnp.cos(ang), jnp.sin(ang)], -1)
    Xhi = X.astype(jnp.bfloat16)
    Xlo = (X - Xhi.astype(jnp.float32)).astype(jnp.bfloat16)
    Xg = _mm(Pf, Xhi) + _mm(Pf, Xlo)
    Xd = jnp.broadcast_to(X[None, :, :], (K, L, 3)).reshape(E, 3)
    ev = Xg - Xd
    d = jnp.sqrt(jnp.sum(ev * ev, -1, keepdims=True) + 1e-8)
    erbf = jnp.exp(-(((d - _rbf_mu()) / _RBF_SIG) ** 2))
    es = jnp.concatenate([erbf, pemb], -1)
    eV = ev / d

    s, V = _gvp_k(embed_node, sin_ref[...], vpk_ref[...])
    es, eV = _gvp_k(embed_edge, es, eV)

    for lyr in layers:
        Wss = lyr['msg0'][3][NS:2 * NS]
        sgt = jnp.dot(Pf, _mm(s, Wss).astype(jnp.bfloat16),
                      preferred_element_type=jnp.float32)
        Vg = jnp.dot(Pf, V.astype(jnp.bfloat16),
                     preferred_element_type=jnp.float32)
        ms, mV = _gvp_msg0(lyr['msg0'], s, V, sgt, Vg, es, eV)
        ms, mV = _gvp_k(lyr['msg1'], ms, mV)
        ms, mV = _gvp_k(lyr['msg2'], ms, mV)
        agg_s = jnp.sum(ms.reshape(K, L, NS), axis=0) * (1.0 / K)
        agg_V = jnp.sum(mV.reshape(K, L, 3 * NV), axis=0) * (1.0 / K)
        s, V = _ln_k(s + agg_s, V + agg_V, lyr['ln1_g'], lyr['ln1_b'])
        fs, fV = _gvp_k(lyr['ff0'], s, V)
        fs, fV = _gvp_k(lyr['ff1'], fs, fV)
        s, V = _ln_k(s + fs, V + fV, lyr['ln2_g'], lyr['ln2_b'])

    out_s_ref[...] = s
    out_v_ref[...] = V


def kernel(coords, coord_mask, padding_mask, confidence, params):
    b, l = coords.shape[0], coords.shape[1]
    X_ca = coords[:, :, 1, :]
    node_s = jnp.concatenate(
        [_dihedrals(coords), _rbf(confidence, 0.0, 1.0, 16)], -1)
    nV = _orientations(X_ca)
    vpk = jnp.concatenate([nV[..., 0], nV[..., 1], nV[..., 2]], -1)
    valid = jnp.logical_and(jnp.logical_not(padding_mask), coord_mask)
    vf = valid.astype(jnp.float32)
    vcol = vf[:, :, None]
    vrow = vf[:, None, :]
    xcat = jnp.swapaxes(X_ca, 1, 2)

    wflat = _flatten_params(params)

    def bspec(width):
        return pl.BlockSpec((None, L, width), lambda i: (i, 0, 0))

    in_specs = [
        bspec(3),
        pl.BlockSpec((None, 3, L), lambda i: (i, 0, 0)),
        bspec(1),
        pl.BlockSpec((None, 1, L), lambda i: (i, 0, 0)),
        bspec(22), bspec(6),
    ]
    for wa in wflat:
        in_specs.append(pl.BlockSpec(wa.shape, lambda i: (0, 0)))

    out_specs = [bspec(NS), bspec(3 * NV)]
    out_shape = [jax.ShapeDtypeStruct((b, l, NS), jnp.float32),
                 jax.ShapeDtypeStruct((b, l, 3 * NV), jnp.float32)]

    s, Vp = pl.pallas_call(
        _fwd_kernel,
        grid=(b,),
        in_specs=in_specs,
        out_specs=out_specs,
        out_shape=out_shape,
        compiler_params=pltpu.CompilerParams(
            vmem_limit_bytes=100 * 1024 * 1024),
    )(X_ca, xcat, vcol, vrow, node_s, vpk, *wflat)

    V = Vp.reshape(b, l, 3, NV).transpose(0, 1, 3, 2).reshape(b, l, NV * 3)
    return jnp.concatenate([s, V], -1)
